# baseline (speedup 1.0000x reference)
import functools

import jax
import jax.numpy as jnp
from jax import lax
from jax.experimental import pallas as pl
from jax.experimental.pallas import tpu as pltpu
from jax.experimental.pallas import tpu_sc as plsc

_B = 16
_L = 2048
_T = 32
_D = 32
_G = _L // _T
_HALF = _L // 2
_GH = 16
_NL = 16
_I32 = jnp.int32
_F32 = jnp.float32


def _sig(x):
    return 1.0 / (1.0 + jnp.exp(-x))


_DNUMS = lax.GatherDimensionNumbers(
    offset_dims=(), collapsed_slice_dims=(0,), start_index_map=(0,))


def _reg_gather(x, idx):
    return lax.gather(x, idx[:, None], _DNUMS, slice_sizes=(1,),
                      mode=lax.GatherScatterMode.PROMISE_IN_BOUNDS)


def _splat_pair(lo, hi, k):
    kv = jnp.broadcast_to(jnp.asarray(k, _I32), (_NL,))
    idx = jnp.bitwise_and(kv, _NL - 1)
    return jnp.where(kv < _NL, _reg_gather(lo, idx), _reg_gather(hi, idx))


def _allsum(x, lanes):
    for sh in (8, 4, 2, 1):
        x = x + _reg_gather(x, jnp.bitwise_xor(lanes, sh))
    return x


def _zz():
    return jnp.zeros((_NL,), _F32)


def _body(a_hbm, v_hbm, hWT_hbm, wWT_hbm, hb_hbm, wb_hbm, conv_hbm,
          oa_hbm, ov_hbm,
          a_v, v_v, sums_v, stats_v, c_v, s_v,
          hWT_v, wWT_v, hb_v, wb_v, conv_v, stats_sh):
    c_id = lax.axis_index("c")
    s_id = lax.axis_index("s")
    i = c_id * 8 + s_id // 2
    i_loc = s_id // 2
    h = s_id % 2
    base = h * _HALF
    inv_g = 1.0 / _G
    inv_d = 1.0 / _D
    third = 1.0 / 3.0
    lanes = lax.iota(_I32, _NL)

    pltpu.sync_copy(a_hbm.at[i, :, pl.ds(base, _HALF)], a_v)
    pltpu.sync_copy(v_hbm.at[i, :, pl.ds(base, _HALF)], v_v)
    pltpu.sync_copy(hWT_hbm, hWT_v)
    pltpu.sync_copy(wWT_hbm, wWT_v)
    pltpu.sync_copy(hb_hbm, hb_v)
    pltpu.sync_copy(wb_hbm, wb_v)
    pltpu.sync_copy(conv_hbm, conv_v)

    def group_sum(src, out_slot):
        def per_row(d, _):
            def per_group(gl, rowsums):
                col = gl * _G
                acc = (src[d, pl.ds(col, _NL)]
                       + src[d, pl.ds(col + _NL, _NL)]
                       + src[d, pl.ds(col + 2 * _NL, _NL)]
                       + src[d, pl.ds(col + 3 * _NL, _NL)])
                rs = _allsum(acc, lanes)
                gv = jnp.broadcast_to(jnp.asarray(gl, _I32), (_NL,))
                return jnp.where(lanes == gv, rs, rowsums)
            rowsums = lax.fori_loop(0, _GH, per_group, _zz())
            sums_v[out_slot, d, pl.ds(0, _NL)] = rowsums
            return 0
        lax.fori_loop(0, _D, per_row, 0)

    group_sum(a_v, 0)
    group_sum(v_v, 1)

    pltpu.sync_copy(sums_v, stats_sh.at[i_loc, h])
    plsc.subcore_barrier()
    pltpu.sync_copy(stats_sh.at[i_loc], stats_v)

    w0 = conv_v[0, pl.ds(0, _NL)]
    w1 = conv_v[1, pl.ds(0, _NL)]
    cb = conv_v[2, pl.ds(0, _NL)]

    def hw_row(d, carry):
        rm_lo, rm_hi, cm_lo, cm_hi = carry
        a_glo = stats_v[0, 0, d, pl.ds(0, _NL)] * inv_g
        a_ghi = stats_v[1, 0, d, pl.ds(0, _NL)] * inv_g
        v_glo = stats_v[0, 1, d, pl.ds(0, _NL)] * inv_g
        v_ghi = stats_v[1, 1, d, pl.ds(0, _NL)] * inv_g
        hwlo = (a_glo + v_glo) * 0.5
        hwhi = (a_ghi + v_ghi) * 0.5
        c_v[d, pl.ds(0, _NL)] = _sig(w0 * a_glo + w1 * v_glo + cb)
        c_v[d, pl.ds(_NL, _NL)] = _sig(w0 * a_ghi + w1 * v_ghi + cb)
        cs = _allsum(hwlo + hwhi, lanes) * inv_d
        dv = jnp.broadcast_to(jnp.asarray(d, _I32), (_NL,))
        cm_lo = jnp.where(lanes == dv, cs, cm_lo)
        cm_hi = jnp.where(lanes == dv - _NL, cs, cm_hi)
        return rm_lo + hwlo, rm_hi + hwhi, cm_lo, cm_hi

    rm_lo, rm_hi, cm_lo, cm_hi = lax.fori_loop(
        0, _D, hw_row, (_zz(), _zz(), _zz(), _zz()))
    rm_lo = rm_lo * inv_d
    rm_hi = rm_hi * inv_d

    def h_acc(k, carry):
        hl, hh = carry
        s = _splat_pair(rm_lo, rm_hi, k)
        return (hl + hWT_v[k, pl.ds(0, _NL)] * s,
                hh + hWT_v[k, pl.ds(_NL, _NL)] * s)

    hl, hh = lax.fori_loop(0, _T, h_acc, (_zz(), _zz()))
    h_lo = _sig(hl + hb_v[pl.ds(0, _NL)])
    h_hi = _sig(hh + hb_v[pl.ds(_NL, _NL)])

    def w_acc(k, carry):
        wl, wh = carry
        s = _splat_pair(cm_lo, cm_hi, k)
        return (wl + wWT_v[k, pl.ds(0, _NL)] * s,
                wh + wWT_v[k, pl.ds(_NL, _NL)] * s)

    wl, wh = lax.fori_loop(0, _T, w_acc, (_zz(), _zz()))
    w_lo = _sig(wl + wb_v[pl.ds(0, _NL)])
    w_hi = _sig(wh + wb_v[pl.ds(_NL, _NL)])

    def s_row(d, _):
        wd = _splat_pair(w_lo, w_hi, d)
        s_v[d, pl.ds(0, _NL)] = (h_lo + wd + c_v[d, pl.ds(0, _NL)]) * third
        s_v[d, pl.ds(_NL, _NL)] = (h_hi + wd + c_v[d, pl.ds(_NL, _NL)]) * third
        return 0

    lax.fori_loop(0, _D, s_row, 0)

    def mul_row(d, _):
        srow_lo = s_v[d, pl.ds(0, _NL)]
        srow_hi = s_v[d, pl.ds(_NL, _NL)]

        def per_group(gl, _):
            sc = _splat_pair(srow_lo, srow_hi, _GH * h + gl)
            col = gl * _G
            for q in range(4):
                cq = col + q * _NL
                a_v[d, pl.ds(cq, _NL)] = a_v[d, pl.ds(cq, _NL)] * sc
                v_v[d, pl.ds(cq, _NL)] = v_v[d, pl.ds(cq, _NL)] * sc
            return 0

        lax.fori_loop(0, _GH, per_group, 0)
        return 0

    lax.fori_loop(0, _D, mul_row, 0)
    pltpu.sync_copy(a_v, oa_hbm.at[i, :, pl.ds(base, _HALF)])
    pltpu.sync_copy(v_v, ov_hbm.at[i, :, pl.ds(base, _HALF)])


def kernel(acoustic_seq, visual_seq, IS_BAG_list, hW, hb, wW, wb, convW,
           convb):
    del IS_BAG_list
    at = jnp.transpose(acoustic_seq, (0, 2, 1))
    vt = jnp.transpose(visual_seq, (0, 2, 1))
    conv = jnp.stack([
        jnp.full((_NL,), convW[0, 0, 0, 0], _F32),
        jnp.full((_NL,), convW[0, 1, 0, 0], _F32),
        jnp.full((_NL,), convb[0], _F32),
    ])
    mesh = plsc.VectorSubcoreMesh(core_axis_name="c", subcore_axis_name="s")
    run = functools.partial(
        pl.kernel,
        mesh=mesh,
        out_type=[
            jax.ShapeDtypeStruct((_B, _D, _L), _F32),
            jax.ShapeDtypeStruct((_B, _D, _L), _F32),
        ],
        scratch_types=[
            pltpu.VMEM((_D, _HALF), _F32),
            pltpu.VMEM((_D, _HALF), _F32),
            pltpu.VMEM((2, _D, _GH), _F32),
            pltpu.VMEM((2, 2, _D, _GH), _F32),
            pltpu.VMEM((_D, _T), _F32),
            pltpu.VMEM((_D, _T), _F32),
            pltpu.VMEM((_T, _T), _F32),
            pltpu.VMEM((_T, _T), _F32),
            pltpu.VMEM((_T,), _F32),
            pltpu.VMEM((_T,), _F32),
            pltpu.VMEM((3, _NL), _F32),
            pltpu.VMEM_SHARED((8, 2, 2, _D, _GH), _F32),
        ],
    )(_body)
    out_a, out_v = run(at, vt, hW.T, wW.T, hb, wb, conv)
    return jnp.transpose(out_a, (0, 2, 1)), jnp.transpose(out_v, (0, 2, 1))


# speedup vs baseline: 3.2781x; 3.2781x over previous
import jax
import jax.numpy as jnp
from jax import lax
from jax.experimental import pallas as pl
from jax.experimental.pallas import tpu as pltpu

_L = 2048
_T = 32
_D = 32
_G = _L // _T
_SB = 4
_RW = _SB * _D


def _body(conv_ref, hWT_ref, hb_ref, wBD_ref, wb_ref, a_ref, v_ref,
          oa_ref, ov_ref, big_scratch):
    del big_scratch
    f32 = jnp.float32
    a = a_ref[...].reshape(_RW, _L)
    v = v_ref[...].reshape(_RW, _L)
    ti = lax.broadcasted_iota(jnp.int32, (_L, _T), 0) // _G
    gi = lax.broadcasted_iota(jnp.int32, (_L, _T), 1)
    Q = (ti == gi).astype(f32)
    A_r = jnp.dot(a, Q) * (1.0 / _G)
    V_r = jnp.dot(v, Q) * (1.0 / _G)
    w0 = conv_ref[0]
    w1 = conv_ref[1]
    cb = conv_ref[2]
    c = jax.nn.sigmoid(w0 * A_r + w1 * V_r + cb)
    hw = (A_r + V_r) * 0.5
    si = lax.broadcasted_iota(jnp.int32, (_SB, _RW), 0)
    ri = lax.broadcasted_iota(jnp.int32, (_SB, _RW), 1) // _D
    E = (si == ri).astype(f32) * (1.0 / _D)
    rm = jnp.dot(E, hw)
    H = jax.nn.sigmoid(jnp.dot(rm, hWT_ref[...]) + hb_ref[...])
    cm = jnp.mean(hw, axis=1, keepdims=True)
    w = jax.nn.sigmoid(jnp.dot(wBD_ref[...], cm) + wb_ref[...])
    MT = (E > 0.0).astype(f32)
    dn_bc = (((0,), (0,)), ((), ()))
    Hb = lax.dot_general(MT, H, dn_bc)
    S = (Hb + w + c) * (1.0 / 3.0)
    ug = lax.broadcasted_iota(jnp.int32, (_T, _L), 0)
    ut = lax.broadcasted_iota(jnp.int32, (_T, _L), 1) // _G
    U = (ug == ut).astype(f32)
    scale = jnp.dot(S, U)
    oa_ref[...] = (a * scale).reshape(_SB, _D, _L)
    ov_ref[...] = (v * scale).reshape(_SB, _D, _L)


def kernel(acoustic_seq, visual_seq, IS_BAG_list, hW, hb, wW, wb, convW,
           convb):
    del IS_BAG_list
    B = acoustic_seq.shape[0]
    at = jnp.transpose(acoustic_seq, (0, 2, 1))
    vt = jnp.transpose(visual_seq, (0, 2, 1))
    conv = jnp.stack([convW[0, 0, 0, 0], convW[0, 1, 0, 0], convb[0]])
    hWT = hW.T
    hb2 = hb.reshape(1, _T)
    wBD = jax.scipy.linalg.block_diag(*([wW] * _SB))
    wb4 = jnp.tile(wb, _SB).reshape(_RW, 1)
    seq_spec = pl.BlockSpec((_SB, _D, _L), lambda i: (i, 0, 0))
    full = lambda *s: pl.BlockSpec(s, lambda i: tuple(0 for _ in s))
    out_a, out_v = pl.pallas_call(
        _body,
        grid=(B // _SB,),
        scratch_shapes=[pltpu.VMEM((10 * 1024 * 1024,), jnp.float32)],
        in_specs=[
            pl.BlockSpec(memory_space=pltpu.SMEM),
            full(_T, _T),
            full(1, _T),
            full(_RW, _RW),
            full(_RW, 1),
            seq_spec,
            seq_spec,
        ],
        out_specs=[seq_spec, seq_spec],
        out_shape=[
            jax.ShapeDtypeStruct((B, _D, _L), jnp.float32),
            jax.ShapeDtypeStruct((B, _D, _L), jnp.float32),
        ],
    )(conv, hWT, hb2, wBD, wb4, at, vt)
    return jnp.transpose(out_a, (0, 2, 1)), jnp.transpose(out_v, (0, 2, 1))


# speedup vs baseline: 3.2791x; 1.0003x over previous
import jax
import jax.numpy as jnp
from jax import lax
from jax.experimental import pallas as pl
from jax.experimental.pallas import tpu as pltpu

_L = 2048
_T = 32
_D = 32
_G = _L // _T
_SB = 4
_RW = _SB * _D


def _body(conv_ref, hWT_ref, hb_ref, wBD_ref, wb_ref, a_ref, v_ref,
          oa_ref, ov_ref):
    f32 = jnp.float32
    a = a_ref[...].reshape(_RW, _L)
    v = v_ref[...].reshape(_RW, _L)
    ti = lax.broadcasted_iota(jnp.int32, (_L, _T), 0) // _G
    gi = lax.broadcasted_iota(jnp.int32, (_L, _T), 1)
    Q = (ti == gi).astype(f32)
    A_r = jnp.dot(a, Q) * (1.0 / _G)
    V_r = jnp.dot(v, Q) * (1.0 / _G)
    w0 = conv_ref[0]
    w1 = conv_ref[1]
    cb = conv_ref[2]
    c = jax.nn.sigmoid(w0 * A_r + w1 * V_r + cb)
    hw = (A_r + V_r) * 0.5
    si = lax.broadcasted_iota(jnp.int32, (_SB, _RW), 0)
    ri = lax.broadcasted_iota(jnp.int32, (_SB, _RW), 1) // _D
    E = (si == ri).astype(f32) * (1.0 / _D)
    rm = jnp.dot(E, hw)
    H = jax.nn.sigmoid(jnp.dot(rm, hWT_ref[...]) + hb_ref[...])
    cm = jnp.mean(hw, axis=1, keepdims=True)
    w = jax.nn.sigmoid(jnp.dot(wBD_ref[...], cm) + wb_ref[...])
    MT = (E > 0.0).astype(f32)
    dn_bc = (((0,), (0,)), ((), ()))
    Hb = lax.dot_general(MT, H, dn_bc)
    S = (Hb + w + c) * (1.0 / 3.0)
    ug = lax.broadcasted_iota(jnp.int32, (_T, _L), 0)
    ut = lax.broadcasted_iota(jnp.int32, (_T, _L), 1) // _G
    U = (ug == ut).astype(f32)
    scale = jnp.dot(S, U)
    oa_ref[...] = (a * scale).reshape(_SB, _D, _L)
    ov_ref[...] = (v * scale).reshape(_SB, _D, _L)


def kernel(acoustic_seq, visual_seq, IS_BAG_list, hW, hb, wW, wb, convW,
           convb):
    del IS_BAG_list
    B = acoustic_seq.shape[0]
    at = jnp.transpose(acoustic_seq, (0, 2, 1))
    vt = jnp.transpose(visual_seq, (0, 2, 1))
    conv = jnp.stack([convW[0, 0, 0, 0], convW[0, 1, 0, 0], convb[0]])
    hWT = hW.T
    hb2 = hb.reshape(1, _T)
    wBD = jax.scipy.linalg.block_diag(*([wW] * _SB))
    wb4 = jnp.tile(wb, _SB).reshape(_RW, 1)
    seq_spec = pl.BlockSpec((_SB, _D, _L), lambda i: (i, 0, 0))
    full = lambda *s: pl.BlockSpec(s, lambda i: tuple(0 for _ in s))
    out_a, out_v = pl.pallas_call(
        _body,
        grid=(B // _SB,),
        in_specs=[
            pl.BlockSpec(memory_space=pltpu.SMEM),
            full(_T, _T),
            full(1, _T),
            full(_RW, _RW),
            full(_RW, 1),
            seq_spec,
            seq_spec,
        ],
        out_specs=[seq_spec, seq_spec],
        out_shape=[
            jax.ShapeDtypeStruct((B, _D, _L), jnp.float32),
            jax.ShapeDtypeStruct((B, _D, _L), jnp.float32),
        ],
    )(conv, hWT, hb2, wBD, wb4, at, vt)
    return jnp.transpose(out_a, (0, 2, 1)), jnp.transpose(out_v, (0, 2, 1))
